# baseline (speedup 1.0000x reference)
import functools

import jax
import jax.numpy as jnp
from jax import lax
from jax.experimental import pallas as pl
from jax.experimental.pallas import tpu as pltpu
from jax.experimental.pallas import tpu_sc as plsc

B = 256
T = 77
D = 768
R = B * T

NUM_CORES = 2
NUM_SUBCORES = 16
NW = NUM_CORES * NUM_SUBCORES
RPW = R // NW
G = 16
NG = RPW // G
TAIL = RPW - NG * G
NPAR = 3
NSLOT = NPAR * G
LANES = 16
NC = D // LANES


def _body(tok_hbm, tab_hbm, pos_hbm, out_hbm,
          idx_all, idxt, stage, pos_v,
          gsemA, gsemB, gsemC, osem0, osem1, psem, tsem,
          *slots):
    wid = lax.axis_index("s") * NUM_CORES + lax.axis_index("c")
    base = wid * RPW
    h_pos = pltpu.async_copy(pos_hbm, pos_v, psem)
    pltpu.sync_copy(tok_hbm.at[pl.ds(base, RPW)], idx_all)

    gsems = (gsemA, gsemB, gsemC)
    osems = (osem0, osem1)

    def fire_group(g, par):
        tokv = idx_all[pl.ds(g * G, G)]
        for i in range(G):
            pltpu.async_copy(
                tab_hbm.at[tokv[i]], slots[par * G + i], gsems[par])

    for g0 in range(NPAR):
        fire_group(g0, g0)
    h_pos.wait()

    def step(g, _):
        for p2 in range(2):
            @pl.when((g >= 2) & (lax.rem(g, 2) == p2))
            def _():
                pltpu.make_async_copy(
                    stage.at[pl.ds(p2 * G, G), :],
                    out_hbm.at[pl.ds(base + (g - 2) * G, G), :],
                    osems[p2]).wait()
        roff = lax.rem(g, 2) * G
        for par in range(NPAR):
            @pl.when(lax.rem(g, NPAR) == par)
            def _():
                for i in range(G):
                    pltpu.make_async_copy(
                        tab_hbm.at[0], slots[par * G + i],
                        gsems[par]).wait()
                for i in range(G):
                    t = lax.rem(g * G + i, T)
                    src = slots[par * G + i]

                    @plsc.parallel_loop(0, NC, unroll=8)
                    def _(c):
                        sl = pl.ds(c * LANES, LANES)
                        stage[roff + i, sl] = src[sl] + pos_v[t, sl]

                @pl.when(g <= NG - 1 - NPAR)
                def _():
                    fire_group_dyn(g + NPAR, par)
        for p2 in range(2):
            @pl.when(lax.rem(g, 2) == p2)
            def _():
                pltpu.async_copy(
                    stage.at[pl.ds(p2 * G, G), :],
                    out_hbm.at[pl.ds(base + g * G, G), :], osems[p2])
        return 0

    def fire_group_dyn(g, par):
        tokv = idx_all[pl.ds(g * G, G)]
        for i in range(G):
            pltpu.async_copy(
                tab_hbm.at[tokv[i]], slots[par * G + i], gsems[par])

    lax.fori_loop(0, NG, step, 0)

    pltpu.sync_copy(tok_hbm.at[pl.ds(base + NG * G, TAIL)], idxt)
    pltpu.make_async_copy(
        stage.at[pl.ds(0, G), :],
        out_hbm.at[pl.ds(base + (NG - 2) * G, G), :], osem0).wait()
    pltpu.async_copy(
        tab_hbm.at[idxt], stage.at[pl.ds(0, TAIL), :], tsem).wait()
    pltpu.make_async_copy(
        stage.at[pl.ds(G, G), :],
        out_hbm.at[pl.ds(base + (NG - 1) * G, G), :], osem1).wait()
    for i in range(TAIL):
        t = lax.rem(NG * G + i, T)
        for c in range(NC):
            sl = pl.ds(c * LANES, LANES)
            stage[G + i, sl] = stage[i, sl] + pos_v[t, sl]
    pltpu.async_copy(
        stage.at[pl.ds(G, TAIL), :],
        out_hbm.at[pl.ds(base + NG * G, TAIL), :], osem1).wait()


def kernel(tokens, token_table, position_embedding):
    tokens_flat = tokens.astype(jnp.int32).reshape(R)

    mesh = plsc.VectorSubcoreMesh(core_axis_name="c", subcore_axis_name="s")
    run = functools.partial(
        pl.kernel,
        out_type=jax.ShapeDtypeStruct((R, D), jnp.float32),
        mesh=mesh,
        scratch_types=(
            [pltpu.VMEM((RPW,), jnp.int32),
             pltpu.VMEM((TAIL,), jnp.int32),
             pltpu.VMEM((2 * G, D), jnp.float32),
             pltpu.VMEM((T, D), jnp.float32),
             pltpu.SemaphoreType.DMA,
             pltpu.SemaphoreType.DMA,
             pltpu.SemaphoreType.DMA,
             pltpu.SemaphoreType.DMA,
             pltpu.SemaphoreType.DMA,
             pltpu.SemaphoreType.DMA,
             pltpu.SemaphoreType.DMA]
            + [pltpu.VMEM((D,), jnp.float32) for _ in range(NSLOT)]
        ),
    )(_body)
    out = run(tokens_flat, token_table, position_embedding)
    return out.reshape(B, T, D)


# speedup vs baseline: 1.0723x; 1.0723x over previous
import functools

import jax
import jax.numpy as jnp
from jax import lax
from jax.experimental import pallas as pl
from jax.experimental.pallas import tpu as pltpu
from jax.experimental.pallas import tpu_sc as plsc

B = 256
T = 77
D = 768
R = B * T

NUM_CORES = 2
NUM_SUBCORES = 16
NW = NUM_CORES * NUM_SUBCORES
RPW = R // NW
CH = 16
NFULL = RPW // CH
TAIL = RPW - NFULL * CH
LANES = 16


def _body(tok_hbm, tab_hbm, pos_hbm, out_hbm,
          idx0, idx1, idxt, g0, g1, o0, o1, pos_v,
          gsem0, gsem1, osem0, osem1, isem0, isem1, psem):
    wid = lax.axis_index("s") * NUM_CORES + lax.axis_index("c")
    base = wid * RPW
    h_pos = pltpu.async_copy(pos_hbm, pos_v, psem)

    idx_b = (idx0, idx1)
    g_b = (g0, g1)
    o_b = (o0, o1)
    gsems = (gsem0, gsem1)
    osems = (osem0, osem1)
    isems = (isem0, isem1)

    def add_chunk(j, src, dst, nrows):
        @plsc.parallel_loop(0, nrows)
        def _(r):
            t = lax.rem(j * CH + r, T)
            for c in range(D // LANES):
                sl = pl.ds(c * LANES, LANES)
                dst[r, sl] = src[r, sl] + pos_v[t, sl]

    pltpu.sync_copy(tok_hbm.at[pl.ds(base, CH)], idx0)
    pltpu.sync_copy(tok_hbm.at[pl.ds(base + CH, CH)], idx1)
    pltpu.async_copy(tab_hbm.at[idx0], g0, gsem0)
    pltpu.async_copy(tab_hbm.at[idx1], g1, gsem1)
    h_pos.wait()

    def step(j, _):
        for b in range(2):
            @pl.when(lax.rem(j, 2) == b)
            def _():
                pltpu.make_async_copy(
                    tab_hbm.at[idx_b[b]], g_b[b], gsems[b]).wait()

                @pl.when(j <= NFULL - 3)
                def _():
                    pltpu.async_copy(
                        tok_hbm.at[pl.ds(base + (j + 2) * CH, CH)],
                        idx_b[b], isems[b])

                @pl.when(j >= 2)
                def _():
                    pltpu.make_async_copy(
                        o_b[b],
                        out_hbm.at[pl.ds(base + (j - 2) * CH, CH), :],
                        osems[b]).wait()

                add_chunk(j, g_b[b], o_b[b], CH)

                @pl.when(j <= NFULL - 3)
                def _():
                    pltpu.make_async_copy(
                        tok_hbm.at[pl.ds(base + (j + 2) * CH, CH)],
                        idx_b[b], isems[b]).wait()
                    pltpu.async_copy(tab_hbm.at[idx_b[b]], g_b[b], gsems[b])

                pltpu.async_copy(
                    o_b[b],
                    out_hbm.at[pl.ds(base + j * CH, CH), :], osems[b])
        return 0

    lax.fori_loop(0, NFULL, step, 0)

    pltpu.sync_copy(tok_hbm.at[pl.ds(base + NFULL * CH, TAIL)], idxt)
    pltpu.async_copy(
        tab_hbm.at[idxt], g0.at[pl.ds(0, TAIL), :], gsem0).wait()
    pltpu.make_async_copy(
        o0, out_hbm.at[pl.ds(base + (NFULL - 2) * CH, CH), :], osem0).wait()
    add_chunk(NFULL, g0, o0, TAIL)
    pltpu.async_copy(
        o0.at[pl.ds(0, TAIL), :],
        out_hbm.at[pl.ds(base + NFULL * CH, TAIL), :], osem0)
    pltpu.make_async_copy(
        o1, out_hbm.at[pl.ds(base + (NFULL - 1) * CH, CH), :], osem1).wait()
    pltpu.make_async_copy(
        o0.at[pl.ds(0, TAIL), :],
        out_hbm.at[pl.ds(base + NFULL * CH, TAIL), :], osem0).wait()


def kernel(tokens, token_table, position_embedding):
    tokens_flat = tokens.astype(jnp.int32).reshape(R)

    mesh = plsc.VectorSubcoreMesh(core_axis_name="c", subcore_axis_name="s")
    run = functools.partial(
        pl.kernel,
        out_type=jax.ShapeDtypeStruct((R, D), jnp.float32),
        mesh=mesh,
        scratch_types=[
            pltpu.VMEM((CH,), jnp.int32),
            pltpu.VMEM((CH,), jnp.int32),
            pltpu.VMEM((TAIL,), jnp.int32),
            pltpu.VMEM((CH, D), jnp.float32),
            pltpu.VMEM((CH, D), jnp.float32),
            pltpu.VMEM((CH, D), jnp.float32),
            pltpu.VMEM((CH, D), jnp.float32),
            pltpu.VMEM((T, D), jnp.float32),
            pltpu.SemaphoreType.DMA,
            pltpu.SemaphoreType.DMA,
            pltpu.SemaphoreType.DMA,
            pltpu.SemaphoreType.DMA,
            pltpu.SemaphoreType.DMA,
            pltpu.SemaphoreType.DMA,
            pltpu.SemaphoreType.DMA,
        ],
    )(_body)
    out = run(tokens_flat, token_table, position_embedding)
    return out.reshape(B, T, D)
